# Initial kernel scaffold; baseline (speedup 1.0000x reference)
import jax
import jax.numpy as jnp
from jax import lax
from jax.experimental import pallas as pl
from jax.experimental.pallas import tpu as pltpu
from jax.experimental.pallas import tpu_sc as plsc

N = 10000
E = 320000
D = 128
LAM = 0.5

NC = 2
NS = 16
NW = NC * NS
EDGES_PER_TILE = E // NW
CHUNK = 80
CHUNKS_PER_TILE = EDGES_PER_TILE // CHUNK
RPT = N // NS

ROW_BLOCK = 1000


def _mlp_body(h_ref, w_ref, b_ref, z_ref):
    z = lax.dot_general(
        h_ref[...], w_ref[...], (((1,), (0,)), ((), ())),
        preferred_element_type=jnp.float32,
        precision=lax.Precision.HIGHEST,
    )
    z = jnp.maximum(z + b_ref[...], 0.0)
    nrm = jnp.sqrt(jnp.sum(z * z, axis=1, keepdims=True))
    z_ref[...] = z / jnp.maximum(nrm, 1e-12)


def _mlp(h, W, b2d):
    return pl.pallas_call(
        _mlp_body,
        grid=(N // ROW_BLOCK,),
        in_specs=[
            pl.BlockSpec((ROW_BLOCK, D), lambda i: (i, 0)),
            pl.BlockSpec((D, D), lambda i: (0, 0)),
            pl.BlockSpec((1, D), lambda i: (0, 0)),
        ],
        out_specs=pl.BlockSpec((ROW_BLOCK, D), lambda i: (i, 0)),
        out_shape=jax.ShapeDtypeStruct((N, D), jnp.float32),
    )(h, W, b2d)


def _make_hop(with_deg):
    mesh = plsc.VectorSubcoreMesh(core_axis_name="c", subcore_axis_name="s")

    out_type = [jax.ShapeDtypeStruct((NC, N, D), jnp.float32)]
    scratch = [
        pltpu.VMEM((CHUNK,), jnp.int32),
        pltpu.VMEM((CHUNK,), jnp.int32),
        pltpu.VMEM((CHUNK, D), jnp.float32),
        pltpu.VMEM_SHARED((N, D), jnp.float32),
    ]
    if with_deg:
        out_type.append(jax.ShapeDtypeStruct((NC, N, 16), jnp.float32))
        scratch += [
            pltpu.VMEM((CHUNK, 16), jnp.float32),
            pltpu.VMEM_SHARED((N, 16), jnp.float32),
        ]

    if with_deg:
        def body(table, src, dst, zrows, zdeg, ones_h,
                 out, degout, idx_s, idx_d, rows, acc, ones_v, deg):
            c = lax.axis_index("c")
            s = lax.axis_index("s")
            w = c * NS + s
            row0 = s * RPT
            pltpu.sync_copy(zrows, acc.at[pl.ds(row0, RPT)])
            pltpu.sync_copy(zdeg, deg.at[pl.ds(row0, RPT)])
            pltpu.sync_copy(ones_h, ones_v)
            plsc.subcore_barrier()

            @pl.loop(0, CHUNKS_PER_TILE)
            def _(i):
                base = pl.multiple_of(w * EDGES_PER_TILE + i * CHUNK, 16)
                pltpu.sync_copy(src.at[pl.ds(base, CHUNK)], idx_s)
                pltpu.sync_copy(dst.at[pl.ds(base, CHUNK)], idx_d)
                pltpu.sync_copy(table.at[idx_s], rows)
                pltpu.sync_copy(rows, acc.at[idx_d], add=True)
                pltpu.sync_copy(ones_v, deg.at[idx_d], add=True)

            plsc.subcore_barrier()
            pltpu.sync_copy(acc.at[pl.ds(row0, RPT)],
                            out.at[c, pl.ds(row0, RPT)])
            pltpu.sync_copy(deg.at[pl.ds(row0, RPT)],
                            degout.at[c, pl.ds(row0, RPT)])
    else:
        def body(table, src, dst, zrows,
                 out, idx_s, idx_d, rows, acc):
            c = lax.axis_index("c")
            s = lax.axis_index("s")
            w = c * NS + s
            row0 = s * RPT
            pltpu.sync_copy(zrows, acc.at[pl.ds(row0, RPT)])
            plsc.subcore_barrier()

            @pl.loop(0, CHUNKS_PER_TILE)
            def _(i):
                base = pl.multiple_of(w * EDGES_PER_TILE + i * CHUNK, 16)
                pltpu.sync_copy(src.at[pl.ds(base, CHUNK)], idx_s)
                pltpu.sync_copy(dst.at[pl.ds(base, CHUNK)], idx_d)
                pltpu.sync_copy(table.at[idx_s], rows)
                pltpu.sync_copy(rows, acc.at[idx_d], add=True)

            plsc.subcore_barrier()
            pltpu.sync_copy(acc.at[pl.ds(row0, RPT)],
                            out.at[c, pl.ds(row0, RPT)])

    return pl.kernel(body, out_type=out_type, mesh=mesh,
                     scratch_types=scratch)


_hop_deg = _make_hop(True)
_hop = _make_hop(False)


def _c1_body(p_ref, pd_ref, out_ref):
    s = p_ref[0] + p_ref[1]
    deg = pd_ref[0, :, 0] + pd_ref[1, :, 0]
    out_ref[...] = s / jnp.maximum(deg, 1.0)[:, None]


def _combine1(p, pdeg):
    return pl.pallas_call(
        _c1_body,
        grid=(N // ROW_BLOCK,),
        in_specs=[
            pl.BlockSpec((NC, ROW_BLOCK, D), lambda i: (0, i, 0)),
            pl.BlockSpec((NC, ROW_BLOCK, 16), lambda i: (0, i, 0)),
        ],
        out_specs=pl.BlockSpec((ROW_BLOCK, D), lambda i: (i, 0)),
        out_shape=jax.ShapeDtypeStruct((N, D), jnp.float32),
    )(p, pdeg)


def _c2_body(n1_ref, p_ref, pd_ref, out_ref):
    s = p_ref[0] + p_ref[1]
    deg = pd_ref[0, :, 0] + pd_ref[1, :, 0]
    neigh2 = s / jnp.maximum(deg, 1.0)[:, None]
    out_ref[...] = LAM * n1_ref[...] + (1.0 - LAM) * neigh2


def _combine2(n1, p, pdeg):
    return pl.pallas_call(
        _c2_body,
        grid=(N // ROW_BLOCK,),
        in_specs=[
            pl.BlockSpec((ROW_BLOCK, D), lambda i: (i, 0)),
            pl.BlockSpec((NC, ROW_BLOCK, D), lambda i: (0, i, 0)),
            pl.BlockSpec((NC, ROW_BLOCK, 16), lambda i: (0, i, 0)),
        ],
        out_specs=pl.BlockSpec((ROW_BLOCK, D), lambda i: (i, 0)),
        out_shape=jax.ShapeDtypeStruct((N, D), jnp.float32),
    )(n1, p, pdeg)


def kernel(h, edge_index, W, b):
    z = _mlp(h, W, b.reshape(1, D))
    src = edge_index[0]
    dst = edge_index[1]
    zrows = jnp.zeros((RPT, D), jnp.float32)
    zdeg = jnp.zeros((RPT, 16), jnp.float32)
    ones = jnp.ones((CHUNK, 16), jnp.float32)
    p1, pdeg = _hop_deg(z, src, dst, zrows, zdeg, ones)
    neigh1 = _combine1(p1, pdeg)
    p2 = _hop(neigh1, src, dst, zrows)
    result = _combine2(neigh1, p2, pdeg)
    return (z, result)


# speedup vs baseline: 5.0826x; 5.0826x over previous
import dataclasses

import jax
import jax.numpy as jnp
from jax import lax
from jax.experimental import pallas as pl
from jax.experimental.pallas import tpu as pltpu
from jax.experimental.pallas import tpu_sc as plsc

N = 10000
E = 320000
D = 128
LAM = 0.5

NC = 2
NS = 16
NW = NC * NS
EDGES_PER_TILE = E // NW
CHUNK = 80
CHUNKS_PER_TILE = EDGES_PER_TILE // CHUNK
ZSTEP = 624
ZSPAN = 640

ROW_BLOCK = 1000


def _mlp_body(h_ref, w_ref, b_ref, z_ref):
    z = lax.dot_general(
        h_ref[...], w_ref[...], (((1,), (0,)), ((), ())),
        preferred_element_type=jnp.float32,
        precision=lax.Precision.HIGHEST,
    )
    z = jnp.maximum(z + b_ref[...], 0.0)
    nrm = jnp.sqrt(jnp.sum(z * z, axis=1, keepdims=True))
    z_ref[...] = z / jnp.maximum(nrm, 1e-12)


def _mlp(h, W, b2d):
    return pl.pallas_call(
        _mlp_body,
        grid=(N // ROW_BLOCK,),
        in_specs=[
            pl.BlockSpec((ROW_BLOCK, D), lambda i: (i, 0)),
            pl.BlockSpec((D, D), lambda i: (0, 0)),
            pl.BlockSpec((1, D), lambda i: (0, 0)),
        ],
        out_specs=pl.BlockSpec((ROW_BLOCK, D), lambda i: (i, 0)),
        out_shape=jax.ShapeDtypeStruct((N, D), jnp.float32),
    )(h, W, b2d)


def _make_hop(with_deg):
    mesh = plsc.VectorSubcoreMesh(core_axis_name="c", subcore_axis_name="s")

    out_type = [jax.ShapeDtypeStruct((NC, N, D), jnp.float32)]
    scratch = [
        pltpu.VMEM((CHUNK,), jnp.int32),
        pltpu.VMEM((CHUNK,), jnp.int32),
        pltpu.VMEM((CHUNK, D), jnp.float32),
        pltpu.VMEM_SHARED((N, D), jnp.float32),
    ]
    if with_deg:
        out_type.append(jax.ShapeDtypeStruct((NW, 8, N), jnp.float32))
        scratch.append(pltpu.VMEM((N,), jnp.float32))

    if with_deg:
        def body(table, src, dst, zrows,
                 out, degout, idx_s, idx_d, rows, acc, degt):
            c = lax.axis_index("c")
            s = lax.axis_index("s")
            w = c * NS + s
            row0 = pl.multiple_of(s * ZSTEP, 8)
            pltpu.sync_copy(zrows, acc.at[pl.ds(row0, ZSPAN)])

            @pl.loop(0, N // 16)
            def _(i):
                degt[pl.ds(pl.multiple_of(i * 16, 16), 16)] = jnp.zeros(
                    (16,), jnp.float32)

            plsc.subcore_barrier()

            @pl.loop(0, CHUNKS_PER_TILE)
            def _(i):
                base = pl.multiple_of(w * EDGES_PER_TILE + i * CHUNK, 16)
                pltpu.sync_copy(src.at[pl.ds(base, CHUNK)], idx_s)
                pltpu.sync_copy(dst.at[pl.ds(base, CHUNK)], idx_d)
                pltpu.sync_copy(table.at[idx_s], rows)
                pltpu.sync_copy(rows, acc.at[idx_d], add=True)
                for j in range(CHUNK // 16):
                    iv = idx_d[pl.ds(j * 16, 16)]
                    plsc.addupdate_scatter(degt, [iv],
                                           jnp.ones((16,), jnp.float32))

            plsc.subcore_barrier()
            pltpu.sync_copy(acc.at[pl.ds(row0, ZSPAN)],
                            out.at[c, pl.ds(row0, ZSPAN)])
            pltpu.sync_copy(degt, degout.at[w, 0])
    else:
        def body(table, src, dst, zrows,
                 out, idx_s, idx_d, rows, acc):
            c = lax.axis_index("c")
            s = lax.axis_index("s")
            w = c * NS + s
            row0 = pl.multiple_of(s * ZSTEP, 8)
            pltpu.sync_copy(zrows, acc.at[pl.ds(row0, ZSPAN)])
            plsc.subcore_barrier()

            @pl.loop(0, CHUNKS_PER_TILE)
            def _(i):
                base = pl.multiple_of(w * EDGES_PER_TILE + i * CHUNK, 16)
                pltpu.sync_copy(src.at[pl.ds(base, CHUNK)], idx_s)
                pltpu.sync_copy(dst.at[pl.ds(base, CHUNK)], idx_d)
                pltpu.sync_copy(table.at[idx_s], rows)
                pltpu.sync_copy(rows, acc.at[idx_d], add=True)

            plsc.subcore_barrier()
            pltpu.sync_copy(acc.at[pl.ds(row0, ZSPAN)],
                            out.at[c, pl.ds(row0, ZSPAN)])

    cp = pltpu.CompilerParams()
    if "needs_layout_passes" in pltpu.CompilerParams.__dataclass_fields__:
        cp = dataclasses.replace(cp, needs_layout_passes=False)
    return pl.kernel(body, out_type=out_type, mesh=mesh,
                     scratch_types=scratch, compiler_params=cp)


_hop_deg = _make_hop(True)
_hop = _make_hop(False)


def _c1_body(p_ref, pd_ref, out_ref):
    s = p_ref[0] + p_ref[1]
    deg = jnp.sum(pd_ref[:, 0, :], axis=0)
    out_ref[...] = s / jnp.maximum(deg, 1.0)[:, None]


def _combine1(p, pdeg):
    return pl.pallas_call(
        _c1_body,
        grid=(1,),
        in_specs=[
            pl.BlockSpec((NC, N, D), lambda i: (0, 0, 0)),
            pl.BlockSpec((NW, 8, N), lambda i: (0, 0, 0)),
        ],
        out_specs=pl.BlockSpec((N, D), lambda i: (0, 0)),
        out_shape=jax.ShapeDtypeStruct((N, D), jnp.float32),
    )(p, pdeg)


def _c2_body(n1_ref, p_ref, pd_ref, out_ref):
    s = p_ref[0] + p_ref[1]
    deg = jnp.sum(pd_ref[:, 0, :], axis=0)
    neigh2 = s / jnp.maximum(deg, 1.0)[:, None]
    out_ref[...] = LAM * n1_ref[...] + (1.0 - LAM) * neigh2


def _combine2(n1, p, pdeg):
    return pl.pallas_call(
        _c2_body,
        grid=(1,),
        in_specs=[
            pl.BlockSpec((N, D), lambda i: (0, 0)),
            pl.BlockSpec((NC, N, D), lambda i: (0, 0, 0)),
            pl.BlockSpec((NW, 8, N), lambda i: (0, 0, 0)),
        ],
        out_specs=pl.BlockSpec((N, D), lambda i: (0, 0)),
        out_shape=jax.ShapeDtypeStruct((N, D), jnp.float32),
    )(n1, p, pdeg)


def kernel(h, edge_index, W, b):
    z = _mlp(h, W, b.reshape(1, D))
    src = edge_index[0]
    dst = edge_index[1]
    zrows = jnp.zeros((ZSPAN, D), jnp.float32)
    p1, pdeg = _hop_deg(z, src, dst, zrows)
    neigh1 = _combine1(p1, pdeg)
    (p2,) = _hop(neigh1, src, dst, zrows)
    result = _combine2(neigh1, p2, pdeg)
    return (z, result)
